# Initial kernel scaffold; baseline (speedup 1.0000x reference)
import jax
import jax.numpy as jnp
from jax import lax
from jax.experimental import pallas as pl
from jax.experimental.pallas import tpu as pltpu
from jax.experimental.pallas import tpu_sc as plsc

D = 32
SCALE = D ** 0.5
G = 128
NG = 10
C = G * NG
NC_ = 2
NS_ = 16
NW = NC_ * NS_


def _sc_embed(idx_hbm, table_hbm, out_hbm, idx_v, rows_v, gsem):
    wid = lax.axis_index("s") * NC_ + lax.axis_index("c")
    n_rows = out_hbm.shape[0]
    b_per_w = n_rows // NW
    n_chunks = b_per_w // C
    base = wid * b_per_w

    def chunk_body(g, carry):
        row0 = base + g * C
        pltpu.sync_copy(idx_hbm.at[pl.ds(row0 // G, NG), :], idx_v)
        handles = []
        for j in range(NG):
            handles.append(pltpu.async_copy(
                table_hbm.at[idx_v.at[j]], rows_v.at[pl.ds(j * G, G)], gsem))
        for h in handles:
            h.wait()

        def scale_body(i, c2):
            for r in range(4):
                row = i * 4 + r
                rows_v[row, 0:16] = rows_v[row, 0:16] * SCALE
                rows_v[row, 16:32] = rows_v[row, 16:32] * SCALE
            return c2
        lax.fori_loop(0, C // 4, scale_body, 0)

        pltpu.sync_copy(rows_v, out_hbm.at[pl.ds(row0, C), :])
        return carry

    lax.fori_loop(0, n_chunks, chunk_body, 0)


def kernel(inputs, lookup_table):
    B = inputs.shape[0] * inputs.shape[1]
    idx = inputs.reshape(B // G, G).astype(jnp.int32)
    mesh = plsc.VectorSubcoreMesh(core_axis_name="c", subcore_axis_name="s")
    out = pl.kernel(
        _sc_embed,
        out_type=jax.ShapeDtypeStruct((B, D), jnp.float32),
        mesh=mesh,
        scratch_types=[
            pltpu.VMEM((NG, G), jnp.int32),
            pltpu.VMEM((C, D), jnp.float32),
            pltpu.SemaphoreType.DMA,
        ],
    )(idx, lookup_table)
    return out.reshape(inputs.shape + (D,))


# speedup vs baseline: 1.4751x; 1.4751x over previous
import jax
import jax.numpy as jnp
from jax import lax
from jax.experimental import pallas as pl
from jax.experimental.pallas import tpu as pltpu
from jax.experimental.pallas import tpu_sc as plsc

D = 32
SCALE = D ** 0.5
G = 128
NG = 8
C = G * NG
NC_ = 2
NS_ = 16
NW = NC_ * NS_


def _sc_embed(idx_hbm, table_hbm, out_hbm, idx_v, rows_v, gsem):
    wid = lax.axis_index("s") * NC_ + lax.axis_index("c")
    n_rows = out_hbm.shape[0]
    b_per_w = n_rows // NW
    n_chunks = b_per_w // C
    base = wid * b_per_w

    def chunk_body(g, carry):
        row0 = pl.multiple_of(base + g * C, C)
        pltpu.sync_copy(idx_hbm.at[pl.ds(pl.multiple_of(row0 // G, NG), NG), :], idx_v)
        handles = []
        for j in range(NG):
            handles.append(pltpu.async_copy(
                table_hbm.at[idx_v.at[j]], rows_v.at[pl.ds(j * G, G)], gsem))
        for h in handles:
            h.wait()

        def scale_body(i, c2):
            for r in range(4):
                row = i * 4 + r
                rows_v[row, 0:16] = rows_v[row, 0:16] * SCALE
                rows_v[row, 16:32] = rows_v[row, 16:32] * SCALE
            return c2
        lax.fori_loop(0, C // 4, scale_body, 0)

        pltpu.sync_copy(rows_v, out_hbm.at[pl.ds(row0, C), :])
        return carry

    lax.fori_loop(0, n_chunks, chunk_body, 0)


def kernel(inputs, lookup_table):
    B = inputs.shape[0] * inputs.shape[1]
    idx = inputs.reshape(B // G, G).astype(jnp.int32)
    mesh = plsc.VectorSubcoreMesh(core_axis_name="c", subcore_axis_name="s")
    out = pl.kernel(
        _sc_embed,
        out_type=jax.ShapeDtypeStruct((B, D), jnp.float32),
        mesh=mesh,
        compiler_params=pltpu.CompilerParams(use_tc_tiling_on_sc=False),
        scratch_types=[
            pltpu.VMEM((NG, G), jnp.int32),
            pltpu.VMEM((C, D), jnp.float32),
            pltpu.SemaphoreType.DMA,
        ],
    )(idx, lookup_table)
    return out.reshape(inputs.shape + (D,))


# speedup vs baseline: 1.5428x; 1.0459x over previous
import jax
import jax.numpy as jnp
from jax import lax
from jax.experimental import pallas as pl
from jax.experimental.pallas import tpu as pltpu
from jax.experimental.pallas import tpu_sc as plsc

D = 32
SCALE = D ** 0.5
G = 128
NG = 8
C = G * NG
NC_ = 2
NS_ = 16
NW = NC_ * NS_


def _sc_embed(idx_hbm, table_hbm, out_hbm, idx_v, rows_v, gsem, osem):
    wid = lax.axis_index("s") * NC_ + lax.axis_index("c")
    n_rows = out_hbm.shape[0]
    b_per_w = n_rows // NW
    n_chunks = b_per_w // C
    base = wid * b_per_w

    def stage_and_fire(c, buf):
        row0 = pl.multiple_of(base + c * C, C)
        pltpu.sync_copy(idx_hbm.at[pl.ds(pl.multiple_of(row0 // G, NG), NG), :],
                        idx_v.at[buf])
        for j in range(NG):
            pltpu.async_copy(table_hbm.at[idx_v.at[buf, j]],
                             rows_v.at[buf, pl.ds(j * G, G)], gsem.at[buf])

    def drain_scale_copy(c, buf):
        for _ in range(NG):
            pltpu.make_async_copy(table_hbm.at[idx_v.at[buf, 0]],
                                  rows_v.at[buf, pl.ds(0, G)], gsem.at[buf]).wait()

        def scale_body(i, carry):
            for r in range(4):
                row = i * 4 + r
                rows_v[buf, row, 0:16] = rows_v[buf, row, 0:16] * SCALE
                rows_v[buf, row, 16:32] = rows_v[buf, row, 16:32] * SCALE
            return carry
        lax.fori_loop(0, C // 4, scale_body, 0)

        row0 = pl.multiple_of(base + c * C, C)
        pltpu.async_copy(rows_v.at[buf], out_hbm.at[pl.ds(row0, C), :],
                         osem.at[buf])

    def wait_out(buf):
        pltpu.make_async_copy(rows_v.at[buf],
                              out_hbm.at[pl.ds(0, C), :], osem.at[buf]).wait()

    stage_and_fire(0, 0)

    def pair_body(p, carry):
        a = 2 * p + 1
        pl.when(p >= 1)(lambda: wait_out(1))
        stage_and_fire(a, 1)
        drain_scale_copy(a - 1, 0)
        wait_out(0)
        stage_and_fire(a + 1, 0)
        drain_scale_copy(a, 1)
        return carry

    lax.fori_loop(0, (n_chunks - 1) // 2, pair_body, 0)

    drain_scale_copy(n_chunks - 1, 0)
    wait_out(1)
    wait_out(0)


def kernel(inputs, lookup_table):
    B = inputs.shape[0] * inputs.shape[1]
    idx = inputs.reshape(B // G, G).astype(jnp.int32)
    mesh = plsc.VectorSubcoreMesh(core_axis_name="c", subcore_axis_name="s")
    out = pl.kernel(
        _sc_embed,
        out_type=jax.ShapeDtypeStruct((B, D), jnp.float32),
        mesh=mesh,
        compiler_params=pltpu.CompilerParams(use_tc_tiling_on_sc=False),
        scratch_types=[
            pltpu.VMEM((2, NG, G), jnp.int32),
            pltpu.VMEM((2, C, D), jnp.float32),
            pltpu.SemaphoreType.DMA((2,)),
            pltpu.SemaphoreType.DMA((2,)),
        ],
    )(idx, lookup_table)
    return out.reshape(inputs.shape + (D,))
